# Initial kernel scaffold; baseline (speedup 1.0000x reference)
import jax
import jax.numpy as jnp
from jax.experimental import pallas as pl


def kernel(x_nchw, w1, b1, gamma, beta, w2, b2):
    raise NotImplementedError("write your pallas kernel here")


# speedup vs baseline: 6.9291x; 6.9291x over previous
import functools

import jax
import jax.numpy as jnp
from jax import lax
from jax.experimental import pallas as pl
from jax.experimental.pallas import tpu as pltpu

_BN_EPS = 1e-5


def _pick_div(n, candidates):
    for c in candidates:
        if n % c == 0:
            return c
    return 1


def _stats_kernel(x_ref, g_ref):
    i = pl.program_id(0)
    xb = x_ref[...]
    ones = jnp.ones((1, xb.shape[1]), jnp.float32)
    xa = jnp.concatenate([xb, ones], axis=0)
    g = lax.dot_general(xa, xa, (((1,), (1,)), ((), ())),
                        preferred_element_type=jnp.float32)

    @pl.when(i == 0)
    def _():
        g_ref[...] = jnp.zeros_like(g_ref)

    g_ref[...] += g


def _main_kernel(x_ref, w1a_ref, w2b_ref, aux_ref, o_ref):
    xb = x_ref[...].astype(jnp.bfloat16)
    ones = jnp.ones((1, xb.shape[1]), jnp.bfloat16)
    xa = jnp.concatenate([xb, ones], axis=0)
    h = jnp.dot(w1a_ref[...], xa,
                preferred_element_type=jnp.float32)
    h = jnp.maximum(h, 0.0).astype(jnp.bfloat16)
    y = jnp.dot(w2b_ref[...], h,
                preferred_element_type=jnp.float32)
    o_ref[...] = jnp.tanh(y + aux_ref[0:1, 0:1])


@jax.jit
def _guide_nn_opt(x_nchw, w1, b1, gamma, beta, w2, b2):
    n, c, hh, ww = x_nchw.shape
    hw = hh * ww
    k = w1.shape[0]
    p = n * hw
    x2d = x_nchw.astype(jnp.float32).reshape(n * c, hw)

    nb1 = _pick_div(n, (32, 16, 8, 4, 2, 1))
    r1 = c * nb1
    t1 = n // nb1
    g_tot = pl.pallas_call(
        _stats_kernel,
        out_shape=jax.ShapeDtypeStruct((r1 + 1, r1 + 1), jnp.float32),
        grid=(t1,),
        in_specs=[pl.BlockSpec((r1, hw), lambda i: (i, 0))],
        out_specs=pl.BlockSpec((r1 + 1, r1 + 1), lambda i: (0, 0)),
        compiler_params=pltpu.CompilerParams(
            dimension_semantics=("arbitrary",)),
    )(x2d)
    gd = g_tot[:r1, :r1].reshape(nb1, c, nb1, c)
    idx = jnp.arange(nb1)
    q = gd[idx, :, idx, :].sum(axis=0)
    s = g_tot[r1, :r1].reshape(nb1, c).sum(axis=0)

    mu = s / p
    cov = q / p - mu[:, None] * mu[None, :]
    mean_y = w1 @ mu[:, None] + b1
    var_y = jnp.sum((w1 @ cov) * w1, axis=1, keepdims=True)

    scale = gamma * lax.rsqrt(var_y + _BN_EPS)
    w1f = w1 * scale
    b1f = scale * (b1 - mean_y) + beta

    nb2 = _pick_div(n, (16, 8, 4, 2, 1))
    t2 = n // nb2
    w1a = jnp.concatenate(
        [jnp.kron(jnp.eye(nb2, dtype=jnp.float32), w1f),
         jnp.tile(b1f, (nb2, 1))],
        axis=1).astype(jnp.bfloat16)
    w2b = jnp.kron(jnp.eye(nb2, dtype=jnp.float32),
                   w2.T).astype(jnp.bfloat16)
    aux = jnp.broadcast_to(b2.astype(jnp.float32), (8, 128))

    out2d = pl.pallas_call(
        _main_kernel,
        out_shape=jax.ShapeDtypeStruct((n, hw), jnp.float32),
        grid=(t2,),
        in_specs=[
            pl.BlockSpec((c * nb2, hw), lambda i: (i, 0)),
            pl.BlockSpec((k * nb2, c * nb2 + 1), lambda i: (0, 0)),
            pl.BlockSpec((nb2, k * nb2), lambda i: (0, 0)),
            pl.BlockSpec((8, 128), lambda i: (0, 0)),
        ],
        out_specs=pl.BlockSpec((nb2, hw), lambda i: (i, 0)),
        compiler_params=pltpu.CompilerParams(
            dimension_semantics=("parallel",)),
    )(x2d, w1a, w2b, aux)

    return out2d.reshape(n, 1, hh, ww)


def kernel(x_nchw, w1, b1, gamma, beta, w2, b2):
    return _guide_nn_opt(x_nchw, w1, b1, gamma, beta, w2, b2)


# speedup vs baseline: 9.0900x; 1.3119x over previous
import functools

import jax
import jax.numpy as jnp
from jax import lax
from jax.experimental import pallas as pl
from jax.experimental.pallas import tpu as pltpu

_BN_EPS = 1e-5


def _pick_tile(p, max_t):
    t = max_t
    while t > 128 and p % t != 0:
        t //= 2
    return t if p % t == 0 else p


def _stats_kernel(x_ref, g_ref):
    i = pl.program_id(0)
    xb = x_ref[...]
    ones = jnp.ones((1, xb.shape[1]), jnp.float32)
    xa = jnp.concatenate([xb, ones], axis=0)
    g = lax.dot_general(xa, xa, (((1,), (1,)), ((), ())),
                        preferred_element_type=jnp.float32)

    @pl.when(i == 0)
    def _():
        g_ref[...] = jnp.zeros_like(g_ref)

    g_ref[...] += g


def _main_kernel(x_ref, w1a_ref, w2b_ref, aux_ref, o_ref):
    xb = x_ref[...].astype(jnp.bfloat16)
    ones = jnp.ones((1, xb.shape[1]), jnp.bfloat16)
    xa = jnp.concatenate([xb, ones], axis=0)
    h = jnp.dot(w1a_ref[...], xa,
                preferred_element_type=jnp.float32)
    h = jnp.maximum(h, 0.0).astype(jnp.bfloat16)
    y = jnp.dot(w2b_ref[...], h,
                preferred_element_type=jnp.float32)
    o_ref[...] = jnp.tanh(y + aux_ref[0:1, 0:1])


@jax.jit
def _guide_nn_opt(x_nchw, w1, b1, gamma, beta, w2, b2):
    n, c, hh, ww = x_nchw.shape
    k = w1.shape[0]
    p = n * hh * ww

    xp = jnp.transpose(x_nchw.astype(jnp.float32),
                       (1, 2, 3, 0)).reshape(c, p)

    t1 = _pick_tile(p, 131072)
    g_tot = pl.pallas_call(
        _stats_kernel,
        out_shape=jax.ShapeDtypeStruct((c + 1, c + 1), jnp.float32),
        grid=(p // t1,),
        in_specs=[pl.BlockSpec((c, t1), lambda i: (0, i))],
        out_specs=pl.BlockSpec((c + 1, c + 1), lambda i: (0, 0)),
        compiler_params=pltpu.CompilerParams(
            dimension_semantics=("arbitrary",)),
    )(xp)

    q = g_tot[:c, :c]
    s = g_tot[c, :c]

    mu = s / p
    cov = q / p - mu[:, None] * mu[None, :]
    mean_y = w1 @ mu[:, None] + b1
    var_y = jnp.sum((w1 @ cov) * w1, axis=1, keepdims=True)

    scale = gamma * lax.rsqrt(var_y + _BN_EPS)
    w1f = w1 * scale
    b1f = scale * (b1 - mean_y) + beta

    t2 = _pick_tile(p, 65536)
    w1a = jnp.concatenate([w1f, b1f], axis=1).astype(jnp.bfloat16)
    w2b = w2.T.astype(jnp.bfloat16)
    aux = jnp.broadcast_to(b2.astype(jnp.float32), (8, 128))

    outp = pl.pallas_call(
        _main_kernel,
        out_shape=jax.ShapeDtypeStruct((1, p), jnp.float32),
        grid=(p // t2,),
        in_specs=[
            pl.BlockSpec((c, t2), lambda i: (0, i)),
            pl.BlockSpec((k, c + 1), lambda i: (0, 0)),
            pl.BlockSpec((1, k), lambda i: (0, 0)),
            pl.BlockSpec((8, 128), lambda i: (0, 0)),
        ],
        out_specs=pl.BlockSpec((1, t2), lambda i: (0, i)),
        compiler_params=pltpu.CompilerParams(
            dimension_semantics=("parallel",)),
    )(xp, w1a, w2b, aux)

    return outp.reshape(1, hh, ww, n).transpose(3, 0, 1, 2)


def kernel(x_nchw, w1, b1, gamma, beta, w2, b2):
    return _guide_nn_opt(x_nchw, w1, b1, gamma, beta, w2, b2)


# speedup vs baseline: 15.4646x; 1.7013x over previous
import functools

import jax
import jax.numpy as jnp
from jax import lax
from jax.experimental import pallas as pl
from jax.experimental.pallas import tpu as pltpu

_BN_EPS = 1e-5


def _pick_bs(hw, max_bs):
    bs = max_bs
    while bs > 8 and hw % bs != 0:
        bs //= 2
    return bs if hw % bs == 0 else hw


def _stats_kernel(x_ref, acc_ref, *, c, bs):
    i = pl.program_id(0)

    @pl.when(i == 0)
    def _():
        acc_ref[...] = jnp.zeros_like(acc_ref)

    xs = [x_ref[j] for j in range(c)]
    planes = xs + [xs[a] * xs[b]
                   for a in range(c) for b in range(a, c)]
    for m, v in enumerate(planes):
        r = v[0:8]
        for j in range(8, bs, 8):
            r = r + v[j:j + 8]
        acc_ref[8 * m:8 * m + 8] += r


def _main_kernel(x_ref, w1a_ref, w2b_ref, aux_ref, o_ref, *, c, bs):
    xv = x_ref[...].reshape(c * bs, x_ref.shape[2])
    ones = jnp.ones((1, xv.shape[1]), jnp.float32)
    xa = jnp.concatenate([xv, ones], axis=0).astype(jnp.bfloat16)
    h = jnp.dot(w1a_ref[...], xa,
                preferred_element_type=jnp.float32)
    r = jnp.maximum(h, 0.0).astype(jnp.bfloat16)
    y = jnp.dot(w2b_ref[...], r,
                preferred_element_type=jnp.float32)
    o_ref[...] = jnp.tanh(y + aux_ref[0:1, 0:1])


@jax.jit
def _guide_nn_opt(x_nchw, w1, b1, gamma, beta, w2, b2):
    n, c, hh, ww = x_nchw.shape
    k = w1.shape[0]
    hw = hh * ww
    p = n * hw

    xp = jnp.transpose(x_nchw.astype(jnp.float32),
                       (1, 2, 3, 0)).reshape(c, hw, n)

    bs1 = _pick_bs(hw, 128)
    nm = c + c * (c + 1) // 2
    acc = pl.pallas_call(
        functools.partial(_stats_kernel, c=c, bs=bs1),
        out_shape=jax.ShapeDtypeStruct((8 * nm, n), jnp.float32),
        grid=(hw // bs1,),
        in_specs=[pl.BlockSpec((c, bs1, n), lambda i: (0, i, 0))],
        out_specs=pl.BlockSpec((8 * nm, n), lambda i: (0, 0)),
        compiler_params=pltpu.CompilerParams(
            dimension_semantics=("arbitrary",)),
    )(xp)

    gv = acc.reshape(nm, 8 * n).sum(axis=1)
    s = gv[:c]
    pairs = {}
    idx = c
    for a in range(c):
        for b in range(a, c):
            pairs[(a, b)] = pairs[(b, a)] = gv[idx]
            idx += 1
    q = jnp.stack([jnp.stack([pairs[(a, b)] for b in range(c)])
                   for a in range(c)])

    mu = s / p
    cov = q / p - mu[:, None] * mu[None, :]
    mean_y = w1 @ mu[:, None] + b1
    var_y = jnp.sum((w1 @ cov) * w1, axis=1, keepdims=True)

    scale = gamma * lax.rsqrt(var_y + _BN_EPS)
    w1f = w1 * scale
    b1f = scale * (b1 - mean_y) + beta

    bs2 = _pick_bs(hw, 64)
    eye = jnp.eye(bs2, dtype=jnp.float32)
    w1a = jnp.concatenate(
        [jnp.kron(w1f, eye), jnp.repeat(b1f, bs2, axis=0)],
        axis=1).astype(jnp.bfloat16)
    w2b = jnp.kron(w2.T, eye).astype(jnp.bfloat16)
    aux = jnp.broadcast_to(b2.astype(jnp.float32), (8, 128))

    outp = pl.pallas_call(
        functools.partial(_main_kernel, c=c, bs=bs2),
        out_shape=jax.ShapeDtypeStruct((hw, n), jnp.float32),
        grid=(hw // bs2,),
        in_specs=[
            pl.BlockSpec((c, bs2, n), lambda i: (0, i, 0)),
            pl.BlockSpec((k * bs2, c * bs2 + 1), lambda i: (0, 0)),
            pl.BlockSpec((bs2, k * bs2), lambda i: (0, 0)),
            pl.BlockSpec((8, 128), lambda i: (0, 0)),
        ],
        out_specs=pl.BlockSpec((bs2, n), lambda i: (i, 0)),
        compiler_params=pltpu.CompilerParams(
            dimension_semantics=("parallel",)),
    )(xp, w1a, w2b, aux)

    return outp.reshape(1, hh, ww, n).transpose(3, 0, 1, 2)


def kernel(x_nchw, w1, b1, gamma, beta, w2, b2):
    return _guide_nn_opt(x_nchw, w1, b1, gamma, beta, w2, b2)
